# baseline (speedup 1.0000x reference)
import functools

import jax
import jax.numpy as jnp
from jax import lax
from jax.experimental import pallas as pl
from jax.experimental.pallas import tpu as pltpu
from jax.experimental.pallas import tpu_sc as plsc

B = 128
N = 8732
NPAD = 8736
FULL = N // 16
TAIL = N - FULL * 16
NWORK = 32
RPW = B // NWORK
NB = 552
NSTEPS = 16


def _lanes():
    return lax.broadcasted_iota(jnp.int32, (16,), 0)


def _ce16(a, b, g):
    s = jnp.where(g == 0, b - a, a - b)
    u = jnp.exp(-jnp.abs(s))
    z = u / (2.0 + u)
    z2 = z * z
    lp = 2.0 * z * (1.0 + z2 * (1.0 / 3.0 + z2 * (
        1.0 / 5.0 + z2 * (1.0 / 7.0 + z2 * (1.0 / 9.0)))))
    return jnp.maximum(s, 0.0) + lp


def _sc_body(pconf_flat, glabel_flat, out_hbm, a_v, b_v, gl_v, closs_v,
             conneg_v, stage_v, sel_s, dma_sem):
    wid = lax.axis_index("s") * 2 + lax.axis_index("c")
    lanes = _lanes()
    zf = jnp.zeros((16,), jnp.float32)
    zi = jnp.zeros((16,), jnp.int32)
    conf_vec = jnp.zeros((16,), jnp.float32)

    for j in range(RPW):
        row = wid * RPW + j
        pbase = pl.multiple_of(row * (2 * N), 8)
        bbase = pl.multiple_of(row * (2 * N) + N - 4, 8)
        graw = row * N
        goff = lax.rem(graw, 8)
        gbase = pl.multiple_of(graw - goff, 8)
        cp_a = pltpu.async_copy(pconf_flat.at[pl.ds(pbase, NPAD)],
                                a_v.at[pl.ds(0, NPAD)], dma_sem)
        cp_b = pltpu.async_copy(pconf_flat.at[pl.ds(bbase, NPAD)],
                                b_v.at[pl.ds(0, NPAD)], dma_sem)
        cp_g = pltpu.async_copy(glabel_flat.at[pl.ds(gbase, NPAD)],
                                gl_v.at[pl.ds(0, NPAD)], dma_sem)
        cp_a.wait()
        cp_b.wait()
        cp_g.wait()

        def loads(i):
            a = a_v[pl.ds(i * 16, 16)]
            b = b_v[pl.ds(i * 16 + 4, 16)]
            g = gl_v[pl.ds(i * 16 + goff, 16)]
            return a, b, g

        def ce_chunk(i, carry):
            tot, pos, npos = carry
            a, b, g = loads(i)
            closs = _ce16(a, b, g)
            posm = g > 0
            return (tot + closs,
                    pos + jnp.where(posm, closs, 0.0),
                    npos + posm.astype(jnp.int32))

        tot, pos, npos = lax.fori_loop(0, FULL, ce_chunk, (zf, zf, zi))
        a, b, g = loads(FULL)
        valid = lanes < TAIL
        closs_t = jnp.where(valid, _ce16(a, b, g), 0.0)
        posm_t = jnp.logical_and(valid, g > 0)
        tot_s = jnp.sum(tot + closs_t)
        pos_s = jnp.sum(pos + jnp.where(posm_t, closs_t, 0.0))
        npos_s = jnp.sum(npos + posm_t.astype(jnp.int32))
        K = jnp.minimum(3 * npos_s, N)

        @pl.when(K >= N)
        def _():
            sel_s[0] = tot_s

        @pl.when(K < N)
        def _():
            def fill_chunk(i, pcnt):
                a, b, g = loads(i)
                closs = _ce16(a, b, g)
                v = jnp.logical_or(i < FULL, lanes < TAIL)
                closs = jnp.where(v, closs, 0.0)
                cn = jnp.where(jnp.logical_or(g > 0, jnp.logical_not(v)),
                               0.0, closs)
                sl = pl.ds(i * 16, 16)
                closs_v[sl] = closs
                conneg_v[sl] = cn
                return pcnt + (cn > 0.0).astype(jnp.int32)

            pcnt_s = jnp.sum(lax.fori_loop(0, FULL + 1, fill_chunk, zi))

            @pl.when(K <= pcnt_s)
            def _():
                def count_gt(t):
                    def cbody(i, acc):
                        bits = plsc.bitcast(conneg_v[pl.ds(i * 16, 16)],
                                            jnp.int32)
                        return acc + (bits > t).astype(jnp.int32)
                    return jnp.sum(lax.fori_loop(0, FULL + 1, cbody, zi))

                def vstep(_, lohi):
                    lo, hi = lohi
                    mid = lo + (hi - lo) // 2
                    take = count_gt(mid) < K
                    return (jnp.where(take, lo, mid + 1),
                            jnp.where(take, mid, hi))

                lo, hi = lax.fori_loop(
                    0, 31, vstep,
                    (jnp.int32(0), jnp.int32((1 << 31) - 1)))
                v = lo

                def gt_sums(i, carry):
                    cnt, sm = carry
                    cn = conneg_v[pl.ds(i * 16, 16)]
                    gt = plsc.bitcast(cn, jnp.int32) > v
                    return (cnt + gt.astype(jnp.int32),
                            sm + jnp.where(gt, cn, 0.0))

                cnt_gt, sum_gt = lax.fori_loop(0, FULL + 1, gt_sums,
                                               (zi, zf))
                cnt_gt_s = jnp.sum(cnt_gt)
                sum_gt_s = jnp.sum(sum_gt)
                thr_vec = plsc.bitcast(jnp.full((16,), v, jnp.int32),
                                       jnp.float32)
                thr_s = jnp.sum(jnp.where(lanes == 0, thr_vec, 0.0))
                ties = (K - cnt_gt_s).astype(jnp.float32)
                sel_s[0] = sum_gt_s + jnp.where(ties > 0, ties * thr_s,
                                                0.0)

            @pl.when(K > pcnt_s)
            def _():
                m = K - pcnt_s

                def count_zlt(p):
                    def cbody(i, acc):
                        cn = conneg_v[pl.ds(i * 16, 16)]
                        idx = i * 16 + lanes
                        ok = jnp.logical_and(cn == 0.0, idx < p)
                        ok = jnp.logical_and(ok, idx < N)
                        return acc + ok.astype(jnp.int32)
                    return jnp.sum(lax.fori_loop(0, FULL + 1, cbody, zi))

                def istep(_, lohi):
                    lo, hi = lohi
                    mid = lo + (hi - lo) // 2
                    take = count_zlt(mid) >= m
                    return (jnp.where(take, lo, mid + 1),
                            jnp.where(take, mid, hi))

                lo, hi = lax.fori_loop(0, 14, istep,
                                       (jnp.int32(0), jnp.int32(N)))
                p = lo

                def zbody(i, acc):
                    sl = pl.ds(i * 16, 16)
                    cn = conneg_v[sl]
                    idx = i * 16 + lanes
                    ok = jnp.logical_and(cn == 0.0, idx < p)
                    ok = jnp.logical_and(ok, idx < N)
                    return acc + jnp.where(ok, closs_v[sl], 0.0)

                zero_part = jnp.sum(lax.fori_loop(0, FULL + 1, zbody, zf))
                sel_s[0] = (tot_s - pos_s) + zero_part

        conf_row = pos_s + sel_s[0]
        nposf = npos_s.astype(jnp.float32)
        conf_vec = jnp.where(lanes == j, conf_row, conf_vec)
        conf_vec = jnp.where(lanes == RPW + j, nposf, conf_vec)

    stage_v[...] = conf_vec
    pltpu.sync_copy(stage_v, out_hbm.at[wid])


def _sc_conf(pconf_flat, glabel_flat):
    mesh = plsc.VectorSubcoreMesh(core_axis_name="c", subcore_axis_name="s")
    kern = functools.partial(
        pl.kernel,
        out_type=jax.ShapeDtypeStruct((NWORK, 16), jnp.float32),
        mesh=mesh,
        scratch_types=[
            pltpu.VMEM((NPAD + 16,), jnp.float32),
            pltpu.VMEM((NPAD + 16,), jnp.float32),
            pltpu.VMEM((NPAD + 16,), jnp.int32),
            pltpu.VMEM((NPAD,), jnp.float32),
            pltpu.VMEM((NPAD,), jnp.float32),
            pltpu.VMEM((16,), jnp.float32),
            pltpu.SMEM((1,), jnp.float32),
            pltpu.SemaphoreType.DMA,
        ],
        compiler_params=pltpu.CompilerParams(needs_layout_passes=False),
    )(_sc_body)
    return kern(pconf_flat, glabel_flat)


def _tc_loc_body(ploc_ref, gloc_ref, glabel_ref, dbox_ref, out_ref):
    step = pl.program_id(0)
    nidx = step * NB + lax.broadcasted_iota(jnp.int32, (NB, B), 0)
    glabel = glabel_ref[...]
    maskf = jnp.where((glabel > 0) & (nidx < N), 1.0, 0.0)

    l1sum = jnp.zeros((NB, B), jnp.float32)
    for c in range(4):
        g = gloc_ref[:, c, :]
        p = ploc_ref[:, c, :]
        d = dbox_ref[:, c, :]
        if c < 2:
            rec = 1.0 / dbox_ref[:, c + 2, :]
            off = (g - d) * rec
        else:
            off = jnp.log(g * (1.0 / d))
        ad = jnp.abs(p - off)
        m = jnp.minimum(ad, 1.0)
        l1sum = l1sum + m * (ad - 0.5 * m)
    part = jnp.sum(maskf * l1sum, axis=0, keepdims=True)

    @pl.when(step == 0)
    def _():
        out_ref[...] = jnp.zeros((1, B), jnp.float32)

    out_ref[...] += part


def _combine_body(loc_ref, conf_ref, npos_ref, out_ref):
    loc = loc_ref[...]
    conf = conf_ref[...]
    nposf = npos_ref[...]
    total = loc + conf
    num_mask = (nposf > 0).astype(jnp.float32)
    scaled = total * num_mask / jnp.maximum(nposf, 1e-6)
    out_ref[...] = jnp.sum(scaled, axis=(0, 1), keepdims=True) * (1.0 / B)


@jax.jit
def kernel(ploc, pconf, gloc, glabel, dboxes):
    glabel32 = glabel.astype(jnp.int32)

    sc_out = _sc_conf(pconf.reshape(-1), glabel32.reshape(-1))

    ploc_phys = jnp.transpose(ploc, (2, 1, 0))
    gloc_phys = jnp.transpose(gloc, (1, 2, 0))
    glab_phys = jnp.transpose(glabel32, (1, 0))
    dbox3 = dboxes.reshape(N, 4, 1)

    loc = pl.pallas_call(
        _tc_loc_body,
        grid=(NSTEPS,),
        in_specs=[
            pl.BlockSpec((NB, 4, B), lambda i: (i, 0, 0)),
            pl.BlockSpec((NB, 4, B), lambda i: (i, 0, 0)),
            pl.BlockSpec((NB, B), lambda i: (i, 0)),
            pl.BlockSpec((NB, 4, 1), lambda i: (i, 0, 0)),
        ],
        out_specs=pl.BlockSpec((1, B), lambda i: (0, 0)),
        out_shape=jax.ShapeDtypeStruct((1, B), jnp.float32),
        compiler_params=pltpu.CompilerParams(
            dimension_semantics=("arbitrary",)),
    )(ploc_phys, gloc_phys, glab_phys, dbox3)

    conf_r = sc_out[:, :RPW].reshape(1, B)
    npos_r = sc_out[:, RPW:2 * RPW].reshape(1, B)

    out = pl.pallas_call(
        _combine_body,
        in_specs=[
            pl.BlockSpec((1, B), lambda: (0, 0)),
            pl.BlockSpec((1, B), lambda: (0, 0)),
            pl.BlockSpec((1, B), lambda: (0, 0)),
        ],
        out_specs=pl.BlockSpec((1, 1), lambda: (0, 0)),
        out_shape=jax.ShapeDtypeStruct((1, 1), jnp.float32),
    )(loc, conf_r, npos_r)
    return out[0, 0]


# speedup vs baseline: 1.2133x; 1.2133x over previous
import functools

import jax
import jax.numpy as jnp
from jax import lax
from jax.experimental import pallas as pl
from jax.experimental.pallas import tpu as pltpu
from jax.experimental.pallas import tpu_sc as plsc

B = 128
N = 8732
NPAD = 8736
FULL = N // 16
TAIL = N - FULL * 16
NWORK = 32
RPW = B // NWORK
NB = 552
NSTEPS = 16


def _lanes():
    return lax.broadcasted_iota(jnp.int32, (16,), 0)


def _ce16(a, b, g):
    s = jnp.where(g == 0, b - a, a - b)
    u = jnp.exp(-jnp.abs(s))
    z = u / (2.0 + u)
    z2 = z * z
    lp = 2.0 * z * (1.0 + z2 * (1.0 / 3.0 + z2 * (
        1.0 / 5.0 + z2 * (1.0 / 7.0 + z2 * (1.0 / 9.0)))))
    return jnp.maximum(s, 0.0) + lp


def _sc_body(pconf_flat, glabel_flat, out_hbm, a_v, b_v, gl_v, closs_v,
             conneg_v, stage_v, sel_s, dma_sem):
    wid = lax.axis_index("s") * 2 + lax.axis_index("c")
    lanes = _lanes()
    zf = jnp.zeros((16,), jnp.float32)
    zi = jnp.zeros((16,), jnp.int32)
    conf_vec = jnp.zeros((16,), jnp.float32)

    for j in range(RPW):
        row = wid * RPW + j
        pbase = pl.multiple_of(row * (2 * N), 8)
        bbase = pl.multiple_of(row * (2 * N) + N - 4, 8)
        graw = row * N
        goff = lax.rem(graw, 8)
        gbase = pl.multiple_of(graw - goff, 8)
        cp_a = pltpu.async_copy(pconf_flat.at[pl.ds(pbase, NPAD)],
                                a_v.at[pl.ds(0, NPAD)], dma_sem)
        cp_b = pltpu.async_copy(pconf_flat.at[pl.ds(bbase, NPAD)],
                                b_v.at[pl.ds(0, NPAD)], dma_sem)
        cp_g = pltpu.async_copy(glabel_flat.at[pl.ds(gbase, NPAD)],
                                gl_v.at[pl.ds(0, NPAD)], dma_sem)
        cp_a.wait()
        cp_b.wait()
        cp_g.wait()

        def loads(i):
            a = a_v[pl.ds(i * 16, 16)]
            b = b_v[pl.ds(i * 16 + 4, 16)]
            g = gl_v[pl.ds(i * 16 + goff, 16)]
            return a, b, g

        def ce_chunk(i, carry):
            tot, pos, npos = carry
            a, b, g = loads(i)
            closs = _ce16(a, b, g)
            posm = g > 0
            return (tot + closs,
                    pos + jnp.where(posm, closs, 0.0),
                    npos + posm.astype(jnp.int32))

        tot, pos, npos = lax.fori_loop(0, FULL, ce_chunk, (zf, zf, zi))
        a, b, g = loads(FULL)
        valid = lanes < TAIL
        closs_t = jnp.where(valid, _ce16(a, b, g), 0.0)
        posm_t = jnp.logical_and(valid, g > 0)
        tot_s = jnp.sum(tot + closs_t)
        pos_s = jnp.sum(pos + jnp.where(posm_t, closs_t, 0.0))
        npos_s = jnp.sum(npos + posm_t.astype(jnp.int32))
        K = jnp.minimum(3 * npos_s, N)

        @pl.when(K >= N)
        def _():
            sel_s[0] = tot_s

        @pl.when(K < N)
        def _():
            def fill_chunk(i, pcnt):
                a, b, g = loads(i)
                closs = _ce16(a, b, g)
                v = jnp.logical_or(i < FULL, lanes < TAIL)
                closs = jnp.where(v, closs, 0.0)
                cn = jnp.where(jnp.logical_or(g > 0, jnp.logical_not(v)),
                               0.0, closs)
                sl = pl.ds(i * 16, 16)
                closs_v[sl] = closs
                conneg_v[sl] = cn
                return pcnt + (cn > 0.0).astype(jnp.int32)

            pcnt_s = jnp.sum(lax.fori_loop(0, FULL + 1, fill_chunk, zi))

            @pl.when(K <= pcnt_s)
            def _():
                def count_gt(t):
                    def cbody(i, acc):
                        bits = plsc.bitcast(conneg_v[pl.ds(i * 16, 16)],
                                            jnp.int32)
                        return acc + (bits > t).astype(jnp.int32)
                    return jnp.sum(lax.fori_loop(0, FULL + 1, cbody, zi))

                def vstep(_, lohi):
                    lo, hi = lohi
                    mid = lo + (hi - lo) // 2
                    take = count_gt(mid) < K
                    return (jnp.where(take, lo, mid + 1),
                            jnp.where(take, mid, hi))

                lo, hi = lax.fori_loop(
                    0, 31, vstep,
                    (jnp.int32(0), jnp.int32((1 << 31) - 1)))
                v = lo

                def gt_sums(i, carry):
                    cnt, sm = carry
                    cn = conneg_v[pl.ds(i * 16, 16)]
                    gt = plsc.bitcast(cn, jnp.int32) > v
                    return (cnt + gt.astype(jnp.int32),
                            sm + jnp.where(gt, cn, 0.0))

                cnt_gt, sum_gt = lax.fori_loop(0, FULL + 1, gt_sums,
                                               (zi, zf))
                cnt_gt_s = jnp.sum(cnt_gt)
                sum_gt_s = jnp.sum(sum_gt)
                thr_vec = plsc.bitcast(jnp.full((16,), v, jnp.int32),
                                       jnp.float32)
                thr_s = jnp.sum(jnp.where(lanes == 0, thr_vec, 0.0))
                ties = (K - cnt_gt_s).astype(jnp.float32)
                sel_s[0] = sum_gt_s + jnp.where(ties > 0, ties * thr_s,
                                                0.0)

            @pl.when(K > pcnt_s)
            def _():
                m = K - pcnt_s

                def count_zlt(p):
                    def cbody(i, acc):
                        cn = conneg_v[pl.ds(i * 16, 16)]
                        idx = i * 16 + lanes
                        ok = jnp.logical_and(cn == 0.0, idx < p)
                        ok = jnp.logical_and(ok, idx < N)
                        return acc + ok.astype(jnp.int32)
                    return jnp.sum(lax.fori_loop(0, FULL + 1, cbody, zi))

                def istep(_, lohi):
                    lo, hi = lohi
                    mid = lo + (hi - lo) // 2
                    take = count_zlt(mid) >= m
                    return (jnp.where(take, lo, mid + 1),
                            jnp.where(take, mid, hi))

                lo, hi = lax.fori_loop(0, 14, istep,
                                       (jnp.int32(0), jnp.int32(N)))
                p = lo

                def zbody(i, acc):
                    sl = pl.ds(i * 16, 16)
                    cn = conneg_v[sl]
                    idx = i * 16 + lanes
                    ok = jnp.logical_and(cn == 0.0, idx < p)
                    ok = jnp.logical_and(ok, idx < N)
                    return acc + jnp.where(ok, closs_v[sl], 0.0)

                zero_part = jnp.sum(lax.fori_loop(0, FULL + 1, zbody, zf))
                sel_s[0] = (tot_s - pos_s) + zero_part

        conf_row = pos_s + sel_s[0]
        nposf = npos_s.astype(jnp.float32)
        conf_vec = jnp.where(lanes == j, conf_row, conf_vec)
        conf_vec = jnp.where(lanes == RPW + j, nposf, conf_vec)

    stage_v[...] = conf_vec
    pltpu.sync_copy(stage_v, out_hbm.at[wid])


def _sc_conf(pconf_flat, glabel_flat):
    mesh = plsc.VectorSubcoreMesh(core_axis_name="c", subcore_axis_name="s")
    kern = functools.partial(
        pl.kernel,
        out_type=jax.ShapeDtypeStruct((NWORK, 16), jnp.float32),
        mesh=mesh,
        scratch_types=[
            pltpu.VMEM((NPAD + 16,), jnp.float32),
            pltpu.VMEM((NPAD + 16,), jnp.float32),
            pltpu.VMEM((NPAD + 16,), jnp.int32),
            pltpu.VMEM((NPAD,), jnp.float32),
            pltpu.VMEM((NPAD,), jnp.float32),
            pltpu.VMEM((16,), jnp.float32),
            pltpu.SMEM((1,), jnp.float32),
            pltpu.SemaphoreType.DMA,
        ],
        compiler_params=pltpu.CompilerParams(needs_layout_passes=False),
    )(_sc_body)
    return kern(pconf_flat, glabel_flat)


def _tc_loc_body(ploc_ref, gloc_ref, glabel_ref, dxy_ref, dden_ref, out_ref):
    step = pl.program_id(0)
    nidx = step * NB + lax.broadcasted_iota(jnp.int32, (NB, B), 0)
    glabel = glabel_ref[...]
    maskf = jnp.where((glabel > 0) & (nidx < N), 1.0, 0.0)
    mask3 = maskf.reshape(NB, 1, B)

    g = gloc_ref[...]
    p = ploc_ref[...]
    rec = 1.0 / dden_ref[...]
    t = g * rec
    iswh = lax.broadcasted_iota(jnp.int32, (NB, 4, 1), 1) >= 2
    off = jnp.where(iswh, jnp.log(t), t - dxy_ref[...] * rec)
    ad = jnp.abs(p - off)
    m = jnp.minimum(ad, 1.0)
    l1 = m * (ad - 0.5 * m)
    part = jnp.sum(mask3 * l1, axis=(0, 1)).reshape(1, B)

    @pl.when(step == 0)
    def _():
        out_ref[...] = jnp.zeros((1, B), jnp.float32)

    out_ref[...] += part


def _combine_body(loc_ref, conf_ref, npos_ref, out_ref):
    loc = loc_ref[...]
    conf = conf_ref[...]
    nposf = npos_ref[...]
    total = loc + conf
    num_mask = (nposf > 0).astype(jnp.float32)
    scaled = total * num_mask / jnp.maximum(nposf, 1e-6)
    out_ref[...] = jnp.sum(scaled, axis=(0, 1), keepdims=True) * (1.0 / B)


@jax.jit
def kernel(ploc, pconf, gloc, glabel, dboxes):
    glabel32 = glabel.astype(jnp.int32)

    sc_out = _sc_conf(pconf.reshape(-1), glabel32.reshape(-1))

    ploc_phys = jnp.transpose(ploc, (2, 1, 0))
    gloc_phys = jnp.transpose(gloc, (1, 2, 0))
    glab_phys = jnp.transpose(glabel32, (1, 0))
    dxy3 = dboxes.reshape(N, 4, 1)
    dden3 = jnp.concatenate([dboxes[:, 2:], dboxes[:, 2:]],
                            axis=1).reshape(N, 4, 1)

    loc = pl.pallas_call(
        _tc_loc_body,
        grid=(NSTEPS,),
        in_specs=[
            pl.BlockSpec((NB, 4, B), lambda i: (i, 0, 0)),
            pl.BlockSpec((NB, 4, B), lambda i: (i, 0, 0)),
            pl.BlockSpec((NB, B), lambda i: (i, 0)),
            pl.BlockSpec((NB, 4, 1), lambda i: (i, 0, 0)),
            pl.BlockSpec((NB, 4, 1), lambda i: (i, 0, 0)),
        ],
        out_specs=pl.BlockSpec((1, B), lambda i: (0, 0)),
        out_shape=jax.ShapeDtypeStruct((1, B), jnp.float32),
        compiler_params=pltpu.CompilerParams(
            dimension_semantics=("arbitrary",)),
    )(ploc_phys, gloc_phys, glab_phys, dxy3, dden3)

    conf_r = sc_out[:, :RPW].reshape(1, B)
    npos_r = sc_out[:, RPW:2 * RPW].reshape(1, B)

    out = pl.pallas_call(
        _combine_body,
        in_specs=[
            pl.BlockSpec((1, B), lambda: (0, 0)),
            pl.BlockSpec((1, B), lambda: (0, 0)),
            pl.BlockSpec((1, B), lambda: (0, 0)),
        ],
        out_specs=pl.BlockSpec((1, 1), lambda: (0, 0)),
        out_shape=jax.ShapeDtypeStruct((1, 1), jnp.float32),
    )(loc, conf_r, npos_r)
    return out[0, 0]


# speedup vs baseline: 1.6484x; 1.3586x over previous
import functools

import jax
import jax.numpy as jnp
from jax import lax
from jax.experimental import pallas as pl
from jax.experimental.pallas import tpu as pltpu
from jax.experimental.pallas import tpu_sc as plsc

B = 128
N = 8732
NPAD = 8736
FULL = N // 16
TAIL = N - FULL * 16
NWORK = 32
RPW = B // NWORK
NB = 552
NSTEPS = 16


def _lanes():
    return lax.broadcasted_iota(jnp.int32, (16,), 0)


def _ce16(a, b, g):
    s = jnp.where(g == 0, b - a, a - b)
    u = jnp.exp(-jnp.abs(s))
    z = u / (2.0 + u)
    z2 = z * z
    lp = 2.0 * z * (1.0 + z2 * (1.0 / 3.0 + z2 * (
        1.0 / 5.0 + z2 * (1.0 / 7.0 + z2 * (1.0 / 9.0)))))
    return jnp.maximum(s, 0.0) + lp


def _sc_body(pconf_flat, glabel_flat, out_hbm, a_v, b_v, gl_v, closs_v,
             conneg_v, stage_v, sel_s, dma_sem):
    wid = lax.axis_index("s") * 2 + lax.axis_index("c")
    lanes = _lanes()
    zf = jnp.zeros((16,), jnp.float32)
    zi = jnp.zeros((16,), jnp.int32)
    conf_vec = jnp.zeros((16,), jnp.float32)

    for j in range(RPW):
        row = wid * RPW + j
        pbase = pl.multiple_of(row * (2 * N), 8)
        bbase = pl.multiple_of(row * (2 * N) + N - 4, 8)
        graw = row * N
        goff = lax.rem(graw, 8)
        gbase = pl.multiple_of(graw - goff, 8)
        cp_a = pltpu.async_copy(pconf_flat.at[pl.ds(pbase, NPAD)],
                                a_v.at[pl.ds(0, NPAD)], dma_sem)
        cp_b = pltpu.async_copy(pconf_flat.at[pl.ds(bbase, NPAD)],
                                b_v.at[pl.ds(0, NPAD)], dma_sem)
        cp_g = pltpu.async_copy(glabel_flat.at[pl.ds(gbase, NPAD)],
                                gl_v.at[pl.ds(0, NPAD)], dma_sem)
        cp_a.wait()
        cp_b.wait()
        cp_g.wait()

        def loads(i):
            a = a_v[pl.ds(i * 16, 16)]
            b = b_v[pl.ds(i * 16 + 4, 16)]
            g = gl_v[pl.ds(i * 16 + goff, 16)]
            return a, b, g

        def ce_chunk(i, carry):
            tot, pos, npos = carry
            a, b, g = loads(i)
            closs = _ce16(a, b, g)
            posm = g > 0
            return (tot + closs,
                    pos + jnp.where(posm, closs, 0.0),
                    npos + posm.astype(jnp.int32))

        tot, pos, npos = lax.fori_loop(0, FULL, ce_chunk, (zf, zf, zi))
        a, b, g = loads(FULL)
        valid = lanes < TAIL
        closs_t = jnp.where(valid, _ce16(a, b, g), 0.0)
        posm_t = jnp.logical_and(valid, g > 0)
        tot_s = jnp.sum(tot + closs_t)
        pos_s = jnp.sum(pos + jnp.where(posm_t, closs_t, 0.0))
        npos_s = jnp.sum(npos + posm_t.astype(jnp.int32))
        K = jnp.minimum(3 * npos_s, N)

        @pl.when(K >= N)
        def _():
            sel_s[0] = tot_s

        @pl.when(K < N)
        def _():
            def fill_chunk(i, pcnt):
                a, b, g = loads(i)
                closs = _ce16(a, b, g)
                v = jnp.logical_or(i < FULL, lanes < TAIL)
                closs = jnp.where(v, closs, 0.0)
                cn = jnp.where(jnp.logical_or(g > 0, jnp.logical_not(v)),
                               0.0, closs)
                sl = pl.ds(i * 16, 16)
                closs_v[sl] = closs
                conneg_v[sl] = cn
                return pcnt + (cn > 0.0).astype(jnp.int32)

            pcnt_s = jnp.sum(lax.fori_loop(0, FULL + 1, fill_chunk, zi))

            @pl.when(K <= pcnt_s)
            def _():
                def count_gt(t):
                    def cbody(i, acc):
                        bits = plsc.bitcast(conneg_v[pl.ds(i * 16, 16)],
                                            jnp.int32)
                        return acc + (bits > t).astype(jnp.int32)
                    return jnp.sum(lax.fori_loop(0, FULL + 1, cbody, zi))

                def vstep(_, lohi):
                    lo, hi = lohi
                    mid = lo + (hi - lo) // 2
                    take = count_gt(mid) < K
                    return (jnp.where(take, lo, mid + 1),
                            jnp.where(take, mid, hi))

                lo, hi = lax.fori_loop(
                    0, 31, vstep,
                    (jnp.int32(0), jnp.int32((1 << 31) - 1)))
                v = lo

                def gt_sums(i, carry):
                    cnt, sm = carry
                    cn = conneg_v[pl.ds(i * 16, 16)]
                    gt = plsc.bitcast(cn, jnp.int32) > v
                    return (cnt + gt.astype(jnp.int32),
                            sm + jnp.where(gt, cn, 0.0))

                cnt_gt, sum_gt = lax.fori_loop(0, FULL + 1, gt_sums,
                                               (zi, zf))
                cnt_gt_s = jnp.sum(cnt_gt)
                sum_gt_s = jnp.sum(sum_gt)
                thr_vec = plsc.bitcast(jnp.full((16,), v, jnp.int32),
                                       jnp.float32)
                thr_s = jnp.sum(jnp.where(lanes == 0, thr_vec, 0.0))
                ties = (K - cnt_gt_s).astype(jnp.float32)
                sel_s[0] = sum_gt_s + jnp.where(ties > 0, ties * thr_s,
                                                0.0)

            @pl.when(K > pcnt_s)
            def _():
                m = K - pcnt_s

                def count_zlt(p):
                    def cbody(i, acc):
                        cn = conneg_v[pl.ds(i * 16, 16)]
                        idx = i * 16 + lanes
                        ok = jnp.logical_and(cn == 0.0, idx < p)
                        ok = jnp.logical_and(ok, idx < N)
                        return acc + ok.astype(jnp.int32)
                    return jnp.sum(lax.fori_loop(0, FULL + 1, cbody, zi))

                def istep(_, lohi):
                    lo, hi = lohi
                    mid = lo + (hi - lo) // 2
                    take = count_zlt(mid) >= m
                    return (jnp.where(take, lo, mid + 1),
                            jnp.where(take, mid, hi))

                lo, hi = lax.fori_loop(0, 14, istep,
                                       (jnp.int32(0), jnp.int32(N)))
                p = lo

                def zbody(i, acc):
                    sl = pl.ds(i * 16, 16)
                    cn = conneg_v[sl]
                    idx = i * 16 + lanes
                    ok = jnp.logical_and(cn == 0.0, idx < p)
                    ok = jnp.logical_and(ok, idx < N)
                    return acc + jnp.where(ok, closs_v[sl], 0.0)

                zero_part = jnp.sum(lax.fori_loop(0, FULL + 1, zbody, zf))
                sel_s[0] = (tot_s - pos_s) + zero_part

        conf_row = pos_s + sel_s[0]
        nposf = npos_s.astype(jnp.float32)
        conf_vec = jnp.where(lanes == j, conf_row, conf_vec)
        conf_vec = jnp.where(lanes == RPW + j, nposf, conf_vec)

    stage_v[...] = conf_vec
    pltpu.sync_copy(stage_v, out_hbm.at[wid])


SLAB = 273


def _sc_fast_body(pconf_pf, glab_pf, out_hbm, slab_v, gl_v, acc_v, stage_v,
                  dma_sem):
    wid = lax.axis_index("s") * 2 + lax.axis_index("c")
    nlo = wid * SLAB
    nn = jnp.minimum(SLAB, N - nlo)

    pb = pl.multiple_of(nlo * (2 * B), 8)
    gb = pl.multiple_of(nlo * B, 8)

    @pl.when(wid < NWORK - 1)
    def _():
        pltpu.async_copy(pconf_pf.at[pl.ds(pb, SLAB * 2 * B)],
                         slab_v.at[pl.ds(0, SLAB * 2 * B)], dma_sem).wait()
        pltpu.async_copy(glab_pf.at[pl.ds(gb, SLAB * B)],
                         gl_v.at[pl.ds(0, SLAB * B)], dma_sem).wait()

    LAST = N - (NWORK - 1) * SLAB

    @pl.when(wid == NWORK - 1)
    def _():
        pltpu.async_copy(pconf_pf.at[pl.ds(pb, LAST * 2 * B)],
                         slab_v.at[pl.ds(0, LAST * 2 * B)], dma_sem).wait()
        pltpu.async_copy(glab_pf.at[pl.ds(gb, LAST * B)],
                         gl_v.at[pl.ds(0, LAST * B)], dma_sem).wait()

    NCH = B // 16

    def nbody(n, carry):
        pbase = n * (2 * B)
        gbase = n * B
        out = []
        for bc in range(NCH):
            tot, pos, cnt = carry[bc], carry[NCH + bc], carry[2 * NCH + bc]
            a = slab_v[pl.ds(pbase + bc * 16, 16)]
            b = slab_v[pl.ds(pbase + B + bc * 16, 16)]
            g = gl_v[pl.ds(gbase + bc * 16, 16)]
            closs = _ce16(a, b, g)
            posm = g > 0
            out.append((tot + closs, pos + jnp.where(posm, closs, 0.0),
                        cnt + jnp.where(posm, 1.0, 0.0)))
        return tuple(x[0] for x in out) + tuple(x[1] for x in out) + \
            tuple(x[2] for x in out)

    zf = jnp.zeros((16,), jnp.float32)
    accs = lax.fori_loop(0, nn, nbody, (zf,) * (3 * NCH))
    for i in range(3 * NCH):
        acc_v[pl.ds(i * 16, 16)] = accs[i]
    pltpu.sync_copy(acc_v, out_hbm.at[wid])


def _sc_partials(pconf_pf, glab_pf):
    mesh = plsc.VectorSubcoreMesh(core_axis_name="c", subcore_axis_name="s")
    kern = functools.partial(
        pl.kernel,
        out_type=jax.ShapeDtypeStruct((NWORK, 3 * B), jnp.float32),
        mesh=mesh,
        scratch_types=[
            pltpu.VMEM((SLAB * 2 * B,), jnp.float32),
            pltpu.VMEM((SLAB * B,), jnp.int32),
            pltpu.VMEM((3 * B,), jnp.float32),
            pltpu.VMEM((16,), jnp.float32),
            pltpu.SemaphoreType.DMA,
        ],
        compiler_params=pltpu.CompilerParams(needs_layout_passes=False),
    )(_sc_fast_body)
    return kern(pconf_pf, glab_pf)


def _sc_conf(pconf_flat, glabel_flat):
    mesh = plsc.VectorSubcoreMesh(core_axis_name="c", subcore_axis_name="s")
    kern = functools.partial(
        pl.kernel,
        out_type=jax.ShapeDtypeStruct((NWORK, 16), jnp.float32),
        mesh=mesh,
        scratch_types=[
            pltpu.VMEM((NPAD + 16,), jnp.float32),
            pltpu.VMEM((NPAD + 16,), jnp.float32),
            pltpu.VMEM((NPAD + 16,), jnp.int32),
            pltpu.VMEM((NPAD,), jnp.float32),
            pltpu.VMEM((NPAD,), jnp.float32),
            pltpu.VMEM((16,), jnp.float32),
            pltpu.SMEM((1,), jnp.float32),
            pltpu.SemaphoreType.DMA,
        ],
        compiler_params=pltpu.CompilerParams(needs_layout_passes=False),
    )(_sc_body)
    return kern(pconf_flat, glabel_flat)


def _tc_loc_body(ploc_ref, gloc_ref, glabel_ref, dxy_ref, dden_ref, out_ref):
    step = pl.program_id(0)
    nidx = step * NB + lax.broadcasted_iota(jnp.int32, (NB, B), 0)
    glabel = glabel_ref[...]
    maskf = jnp.where((glabel > 0) & (nidx < N), 1.0, 0.0)
    mask3 = maskf.reshape(NB, 1, B)

    g = gloc_ref[...]
    p = ploc_ref[...]
    rec = 1.0 / dden_ref[...]
    t = g * rec
    iswh = lax.broadcasted_iota(jnp.int32, (NB, 4, 1), 1) >= 2
    off = jnp.where(iswh, jnp.log(t), t - dxy_ref[...] * rec)
    ad = jnp.abs(p - off)
    m = jnp.minimum(ad, 1.0)
    l1 = m * (ad - 0.5 * m)
    part = jnp.sum(mask3 * l1, axis=(0, 1)).reshape(1, B)

    @pl.when(step == 0)
    def _():
        out_ref[...] = jnp.zeros((1, B), jnp.float32)

    out_ref[...] += part


def _combine_body(loc_ref, conf_ref, npos_ref, out_ref):
    loc = loc_ref[...]
    conf = conf_ref[...]
    nposf = npos_ref[...]
    total = loc + conf
    num_mask = (nposf > 0).astype(jnp.float32)
    scaled = total * num_mask / jnp.maximum(nposf, 1e-6)
    out_ref[...] = jnp.sum(scaled, axis=(0, 1), keepdims=True) * (1.0 / B)


@jax.jit
def kernel(ploc, pconf, gloc, glabel, dboxes):
    glabel32 = glabel.astype(jnp.int32)

    pconf_pf = jnp.transpose(pconf, (2, 1, 0)).reshape(-1)
    glab_pf = jnp.transpose(glabel32, (1, 0)).reshape(-1)
    part = _sc_partials(pconf_pf, glab_pf)
    tot_g = jnp.sum(part[:, 0:B], axis=0, keepdims=True)
    pos_g = jnp.sum(part[:, B:2 * B], axis=0, keepdims=True)
    npos_g = jnp.sum(part[:, 2 * B:], axis=0, keepdims=True)
    slow = jnp.any(3.0 * npos_g < float(N))

    def _slow_conf():
        sc_out = _sc_conf(pconf.reshape(-1), glabel32.reshape(-1))
        return sc_out[:, :RPW].reshape(1, B)

    conf_r = lax.cond(slow, _slow_conf, lambda: pos_g + tot_g)

    ploc_phys = jnp.transpose(ploc, (2, 1, 0))
    gloc_phys = jnp.transpose(gloc, (1, 2, 0))
    glab_phys = jnp.transpose(glabel32, (1, 0))
    dxy3 = dboxes.reshape(N, 4, 1)
    dden3 = jnp.concatenate([dboxes[:, 2:], dboxes[:, 2:]],
                            axis=1).reshape(N, 4, 1)

    loc = pl.pallas_call(
        _tc_loc_body,
        grid=(NSTEPS,),
        in_specs=[
            pl.BlockSpec((NB, 4, B), lambda i: (i, 0, 0)),
            pl.BlockSpec((NB, 4, B), lambda i: (i, 0, 0)),
            pl.BlockSpec((NB, B), lambda i: (i, 0)),
            pl.BlockSpec((NB, 4, 1), lambda i: (i, 0, 0)),
            pl.BlockSpec((NB, 4, 1), lambda i: (i, 0, 0)),
        ],
        out_specs=pl.BlockSpec((1, B), lambda i: (0, 0)),
        out_shape=jax.ShapeDtypeStruct((1, B), jnp.float32),
        compiler_params=pltpu.CompilerParams(
            dimension_semantics=("arbitrary",)),
    )(ploc_phys, gloc_phys, glab_phys, dxy3, dden3)

    out = pl.pallas_call(
        _combine_body,
        in_specs=[
            pl.BlockSpec((1, B), lambda: (0, 0)),
            pl.BlockSpec((1, B), lambda: (0, 0)),
            pl.BlockSpec((1, B), lambda: (0, 0)),
        ],
        out_specs=pl.BlockSpec((1, 1), lambda: (0, 0)),
        out_shape=jax.ShapeDtypeStruct((1, 1), jnp.float32),
    )(loc, conf_r, npos_g)
    return out[0, 0]
